# baseline (speedup 1.0000x reference)
import functools

import jax
import jax.numpy as jnp
import numpy as np
from jax.experimental import pallas as pl
from jax.experimental.pallas import tpu as pltpu
from jax.experimental.pallas import tpu_sc as plsc

B = 256
K = 10000
K_PAD = 10240
J = 21
G = 8
JD = J * G
HIDDEN = 512
NUM_NEIGH = 5
BIG = 1.0e9

NC = 2
NS = 16
NW = NC * NS
RPW = B // NW
NCHUNK = K_PAD // 16

_A0 = 1.56956466
_A1 = -0.20056456
_A2 = 0.04599389
_HALF_PI = 1.5707963267948966
_SIGN = np.int32(-2147483648)
_ABSM = np.int32(0x7FFFFFFF)


def _dist_mlp_kernel(poseT_ref, trainT_ref, grp_ref,
                     W0_ref, b0_ref, W1_ref, b1_ref, W2_ref, b2_ref,
                     W3_ref, b3_ref, dist_ref, pred_ref):
    poseT = poseT_ref[:]
    p2 = poseT * poseT
    n2 = jax.lax.dot_general(grp_ref[:], p2, (((1,), (0,)), ((), ())),
                             preferred_element_type=jnp.float32)
    pn = poseT * jax.lax.rsqrt(jnp.maximum(n2, 1e-24))

    pnb = pn.astype(jnp.bfloat16)
    trainb = trainT_ref[:].astype(jnp.bfloat16)
    acc = jnp.zeros((B, K_PAD), jnp.float32)
    for j in range(J):
        pj = pnb[G * j:G * (j + 1), :]
        tj = trainb[G * j:G * (j + 1), :]
        dots = jax.lax.dot_general(pj, tj, (((0,), (0,)), ((), ())),
                                   preferred_element_type=jnp.float32)
        xi = jax.lax.bitcast_convert_type(dots, jnp.int32)
        sgn = jax.lax.bitwise_and(xi, _SIGN)
        ax = jax.lax.bitcast_convert_type(
            jax.lax.bitwise_and(xi, _ABSM), jnp.float32)
        t = jnp.maximum(1.0 - ax, 1e-6)
        p = _A0 + ax * (_A1 + ax * _A2)
        u = _HALF_PI - jnp.sqrt(t) * p
        v = jax.lax.bitcast_convert_type(
            jax.lax.bitwise_xor(jax.lax.bitcast_convert_type(u, jnp.int32),
                                sgn), jnp.float32)
        acc = acc + v

    lane = jax.lax.broadcasted_iota(jnp.int32, (B, K_PAD), 1)
    dist_ref[:, :] = jnp.where(lane < K, (J * _HALF_PI) * 0.5 - acc * 0.5, BIG)

    h = jax.lax.dot_general(pn, W0_ref[:], (((0,), (0,)), ((), ())),
                            preferred_element_type=jnp.float32) + b0_ref[:]
    h = jnp.maximum(h, 0.0)
    h = jax.lax.dot_general(h, W1_ref[:], (((1,), (0,)), ((), ())),
                            preferred_element_type=jnp.float32) + b1_ref[:]
    h = jnp.maximum(h, 0.0)
    h = jax.lax.dot_general(h, W2_ref[:], (((1,), (0,)), ((), ())),
                            preferred_element_type=jnp.float32) + b2_ref[:]
    h = jnp.maximum(h, 0.0)
    pred_ref[:, :] = jax.lax.dot_general(
        h, W3_ref[:], (((1,), (0,)), ((), ())),
        preferred_element_type=jnp.float32) + b3_ref[:]


_GDN = jax.lax.GatherDimensionNumbers(
    offset_dims=(), collapsed_slice_dims=(0,), start_index_map=(0,))


def _shuf(v, lanes, sh):
    idx = jax.lax.bitwise_and(lanes + sh, np.int32(15))
    return jax.lax.gather(v, idx[:, None], _GDN, (1,),
                          mode=jax.lax.GatherScatterMode.PROMISE_IN_BOUNDS)


def _allmin(v, lanes):
    for sh in (8, 4, 2, 1):
        v = jnp.minimum(v, _shuf(v, lanes, sh))
    return v


def _sc_top5_body(dist_hbm, out_hbm, rows_v, res_v):
    c = jax.lax.axis_index("c")
    s = jax.lax.axis_index("s")
    wid = s * NC + c
    base = wid * (RPW * K_PAD)
    pltpu.sync_copy(dist_hbm.at[pl.ds(base, RPW * K_PAD)], rows_v)
    lanes = jax.lax.iota(jnp.int32, 16)
    res_vec = jnp.zeros((16,), jnp.float32)
    for r in range(RPW):
        init = tuple(jnp.full((16,), BIG, jnp.float32)
                     for _ in range(NUM_NEIGH))

        def body(i, carry, r=r):
            start = pl.multiple_of(r * K_PAD + i * 16, 16)
            v = rows_v[pl.ds(start, 16)]
            out = []
            for bvec in carry:
                lo = jnp.minimum(bvec, v)
                v = jnp.maximum(bvec, v)
                out.append(lo)
            return tuple(out)

        bs = list(jax.lax.fori_loop(0, NCHUNK, body, init))
        total = jnp.zeros((16,), jnp.float32)
        for _ in range(NUM_NEIGH):
            mval = _allmin(bs[0], lanes)
            total = total + mval
            cand = jnp.where(bs[0] == mval, lanes, np.int32(16))
            flane = _allmin(cand, lanes)
            bs[0] = jnp.where(jnp.logical_and(lanes == flane, bs[0] == mval),
                              BIG, bs[0])
            for i in range(NUM_NEIGH - 1):
                lo = jnp.minimum(bs[i], bs[i + 1])
                hi = jnp.maximum(bs[i], bs[i + 1])
                bs[i], bs[i + 1] = lo, hi
        res_vec = jnp.where(lanes == r, total * (1.0 / NUM_NEIGH), res_vec)
    res_v[...] = res_vec
    pltpu.sync_copy(res_v.at[pl.ds(0, RPW)], out_hbm.at[pl.ds(wid * RPW, RPW)])


def _sc_top5(dist_flat):
    mesh = plsc.VectorSubcoreMesh(core_axis_name="c", subcore_axis_name="s")
    f = functools.partial(
        pl.kernel,
        mesh=mesh,
        out_type=jax.ShapeDtypeStruct((B,), jnp.float32),
        scratch_types=[
            pltpu.VMEM((RPW * K_PAD,), jnp.float32),
            pltpu.VMEM((16,), jnp.float32),
        ],
    )(_sc_top5_body)
    return f(dist_flat)


def _loss_kernel(pred_ref, dv_ref, out_ref):
    out_ref[:, :] = jnp.sum(jnp.abs(pred_ref[:, 0:1] - dv_ref[:, 0:1]),
                            keepdims=True) * (1.0 / B)


def _pad_joint_rows(x):
    j, d, n = x.shape
    return jnp.concatenate(
        [x, jnp.zeros((j, G - d, n), x.dtype)], axis=1).reshape(j * G, n)


def kernel(pose, train_poses, W0, b0, W1, b1, W2, b2, W3, b3):
    poseT = _pad_joint_rows(pose.transpose(1, 2, 0))
    trainT = _pad_joint_rows(train_poses.transpose(1, 2, 0))
    trainT = jnp.concatenate(
        [trainT, jnp.zeros((JD, K_PAD - K), trainT.dtype)], axis=1)
    grp = jnp.kron(jnp.eye(J, dtype=jnp.float32),
                   jnp.ones((G, G), jnp.float32))
    W0p = _pad_joint_rows(W0.reshape(J, 4, HIDDEN))

    dist, pred = pl.pallas_call(
        _dist_mlp_kernel,
        out_shape=(jax.ShapeDtypeStruct((B, K_PAD), jnp.float32),
                   jax.ShapeDtypeStruct((B, 1), jnp.float32)),
    )(poseT, trainT, grp,
      W0p, b0.reshape(1, HIDDEN), W1, b1.reshape(1, HIDDEN),
      W2, b2.reshape(1, HIDDEN), W3, b3.reshape(1, 1))

    dist_vals = _sc_top5(dist.reshape(B * K_PAD))

    out = pl.pallas_call(
        _loss_kernel,
        out_shape=jax.ShapeDtypeStruct((1, 1), jnp.float32),
    )(pred, dist_vals.reshape(B, 1))
    return out.reshape(())


# speedup vs baseline: 1.6319x; 1.6319x over previous
import jax
import jax.numpy as jnp
import numpy as np
from jax.experimental import pallas as pl

B = 256
K = 10000
K_PAD = 10240
J = 21
G = 8
JD = J * G
HIDDEN = 512
NUM_NEIGH = 5
BIG = 1.0e9

_A0 = 1.56956466
_A1 = -0.20056456
_A2 = 0.04599389
_HALF_PI = 1.5707963267948966
_SIGN = np.int32(-2147483648)
_ABSM = np.int32(0x7FFFFFFF)


def _fused_kernel(poseT_ref, trainT_ref, grp_ref,
                  W0_ref, b0_ref, W1_ref, b1_ref, W2_ref, b2_ref,
                  W3_ref, b3_ref, out_ref):
    poseT = poseT_ref[:]
    p2 = poseT * poseT
    n2 = jax.lax.dot_general(grp_ref[:], p2, (((1,), (0,)), ((), ())),
                             preferred_element_type=jnp.float32)
    pn = poseT * jax.lax.rsqrt(jnp.maximum(n2, 1e-24))

    pnb = pn.astype(jnp.bfloat16)
    trainb = trainT_ref[:]
    acc = jnp.zeros((B, K_PAD), jnp.float32)
    for j in range(J):
        pj = pnb[G * j:G * (j + 1), :]
        tj = trainb[G * j:G * (j + 1), :]
        dots = jax.lax.dot_general(pj, tj, (((0,), (0,)), ((), ())),
                                   preferred_element_type=jnp.float32)
        xi = jax.lax.bitcast_convert_type(dots, jnp.int32)
        sgn = jax.lax.bitwise_and(xi, _SIGN)
        ax = jax.lax.bitcast_convert_type(
            jax.lax.bitwise_and(xi, _ABSM), jnp.float32)
        t = jnp.maximum(1.0 - ax, 1e-6)
        p = _A0 + ax * (_A1 + ax * _A2)
        u = _HALF_PI - (t * jax.lax.rsqrt(t)) * p
        v = jax.lax.bitcast_convert_type(
            jax.lax.bitwise_xor(jax.lax.bitcast_convert_type(u, jnp.int32),
                                sgn), jnp.float32)
        acc = acc + v

    lane = jax.lax.broadcasted_iota(jnp.int32, (B, K_PAD), 1)
    dist = jnp.where(lane < K, (J * _HALF_PI) * 0.5 - acc * 0.5, BIG)

    total = jnp.zeros((B, 1), jnp.float32)
    for _ in range(NUM_NEIGH):
        m = jnp.min(dist, axis=1, keepdims=True)
        total = total + m
        hit = jnp.where(dist == m, lane, K_PAD)
        first = jnp.min(hit, axis=1, keepdims=True)
        dist = jnp.where(lane == first, BIG, dist)
    dist_vals = total * (1.0 / NUM_NEIGH)

    h = jax.lax.dot_general(pn, W0_ref[:], (((0,), (0,)), ((), ())),
                            preferred_element_type=jnp.float32) + b0_ref[:]
    h = jnp.maximum(h, 0.0)
    h = jax.lax.dot_general(h, W1_ref[:], (((1,), (0,)), ((), ())),
                            preferred_element_type=jnp.float32) + b1_ref[:]
    h = jnp.maximum(h, 0.0)
    h = jax.lax.dot_general(h, W2_ref[:], (((1,), (0,)), ((), ())),
                            preferred_element_type=jnp.float32) + b2_ref[:]
    h = jnp.maximum(h, 0.0)
    pred = jax.lax.dot_general(h, W3_ref[:], (((1,), (0,)), ((), ())),
                               preferred_element_type=jnp.float32) + b3_ref[:]

    loss = jnp.sum(jnp.abs(pred[:, 0:1] - dist_vals), keepdims=True) * (1.0 / B)
    out_ref[:, :] = loss


def _pad_joint_rows(x):
    j, d, n = x.shape
    return jnp.concatenate(
        [x, jnp.zeros((j, G - d, n), x.dtype)], axis=1).reshape(j * G, n)


def kernel(pose, train_poses, W0, b0, W1, b1, W2, b2, W3, b3):
    poseT = _pad_joint_rows(pose.transpose(1, 2, 0))
    trainT = _pad_joint_rows(train_poses.transpose(1, 2, 0))
    trainT = jnp.concatenate(
        [trainT, jnp.zeros((JD, K_PAD - K), trainT.dtype)],
        axis=1).astype(jnp.bfloat16)
    grp = jnp.kron(jnp.eye(J, dtype=jnp.float32),
                   jnp.ones((G, G), jnp.float32))
    W0p = _pad_joint_rows(W0.reshape(J, 4, HIDDEN))

    out = pl.pallas_call(
        _fused_kernel,
        out_shape=jax.ShapeDtypeStruct((1, 1), jnp.float32),
    )(poseT, trainT, grp,
      W0p, b0.reshape(1, HIDDEN), W1, b1.reshape(1, HIDDEN),
      W2, b2.reshape(1, HIDDEN), W3, b3.reshape(1, 1))
    return out.reshape(())
